# baseline (speedup 1.0000x reference)
import functools

import jax
import jax.numpy as jnp
from jax import lax
from jax.experimental import pallas as pl
from jax.experimental.pallas import tpu as pltpu
from jax.experimental.pallas import tpu_sc as plsc

_NC = 2
_NS = 16
_NW = _NC * _NS
_CHUNK = 128


def _canonicalize_duplicates(idx):
    b = idx.shape[0]
    order = jnp.argsort(idx, stable=True)
    si = idx[order]
    pos = jnp.arange(b, dtype=jnp.int32)
    is_end = jnp.concatenate([si[1:] != si[:-1], jnp.ones((1,), jnp.bool_)])
    winner_sorted = lax.associative_scan(
        jnp.minimum, jnp.where(is_end, pos, b), reverse=True)
    winner_orig = order[winner_sorted].astype(jnp.int32)
    src = jnp.zeros((b,), jnp.int32).at[order].set(winner_orig)
    return src


def _copy_body(src, dst, sem):
    cp = pltpu.make_async_copy(src, dst, sem)
    cp.start()
    cp.wait()


def _make_copy(m, d):
    return pl.pallas_call(
        _copy_body,
        out_shape=jax.ShapeDtypeStruct((m, d), jnp.float32),
        in_specs=[pl.BlockSpec(memory_space=pltpu.MemorySpace.HBM)],
        out_specs=pl.BlockSpec(memory_space=pltpu.MemorySpace.HBM),
        scratch_shapes=[pltpu.SemaphoreType.DMA],
    )


def _make_scatter(d, k):
    mesh = plsc.VectorSubcoreMesh(
        core_axis_name="c", subcore_axis_name="s",
        num_cores=_NC, num_subcores=_NS)

    @functools.partial(
        pl.kernel,
        mesh=mesh,
        compiler_params=pltpu.CompilerParams(use_tc_tiling_on_sc=False),
        scratch_types=[
            pltpu.VMEM((k, _CHUNK), jnp.int32),
            pltpu.VMEM((k, _CHUNK), jnp.int32),
            pltpu.VMEM((_CHUNK, d), jnp.float32),
            pltpu.SemaphoreType.DMA,
            pltpu.SemaphoreType.DMA,
        ],
    )
    def scatter(out_ref, dst_hbm, src_hbm, new_hbm, didx, sidx, rows, gsem, ssem):
        wid = lax.axis_index("s") * _NC + lax.axis_index("c")
        pltpu.sync_copy(dst_hbm.at[wid], didx)
        pltpu.sync_copy(src_hbm.at[wid], sidx)
        for j in range(k):
            pltpu.async_copy(new_hbm.at[sidx.at[j]], rows, gsem).wait()
            pltpu.async_copy(rows, out_ref.at[didx.at[j]], ssem).wait()

    return scatter


def kernel(mat_orig, indices, mat_new):
    m, d = mat_orig.shape
    b = indices.shape[0]
    k = b // (_NW * _CHUNK)

    idx = indices.astype(jnp.int32)
    src = _canonicalize_duplicates(idx)
    dst3 = idx.reshape(_NW, k, _CHUNK)
    src3 = src.reshape(_NW, k, _CHUNK)

    copied = _make_copy(m, d)(mat_orig)
    ref = jax.new_ref(copied)
    _make_scatter(d, k)(ref, dst3, src3, mat_new)
    return ref[...]


# speedup vs baseline: 1.8849x; 1.8849x over previous
import functools

import jax
import jax.numpy as jnp
from jax import lax
from jax.experimental import pallas as pl
from jax.experimental.pallas import tpu as pltpu
from jax.experimental.pallas import tpu_sc as plsc

_NC = 2
_NS = 16
_NW = _NC * _NS
_CHUNK = 128

_SC_PARAMS = pltpu.CompilerParams(use_tc_tiling_on_sc=False)


def _canonicalize_duplicates(idx):
    b = idx.shape[0]
    order = jnp.argsort(idx, stable=True)
    si = idx[order]
    pos = jnp.arange(b, dtype=jnp.int32)
    is_end = jnp.concatenate([si[1:] != si[:-1], jnp.ones((1,), jnp.bool_)])
    winner_sorted = lax.associative_scan(
        jnp.minimum, jnp.where(is_end, pos, b), reverse=True)
    winner_orig = order[winner_sorted].astype(jnp.int32)
    src = jnp.zeros((b,), jnp.int32).at[order].set(winner_orig)
    return src


def _mesh():
    return plsc.VectorSubcoreMesh(
        core_axis_name="c", subcore_axis_name="s",
        num_cores=_NC, num_subcores=_NS)


def _make_copy(m, d):
    rows_per_w = (m // _NW) // 8 * 8
    tail = m - rows_per_w * _NW

    @functools.partial(
        pl.kernel,
        mesh=_mesh(),
        compiler_params=_SC_PARAMS,
        scratch_types=[pltpu.SemaphoreType.DMA],
    )
    def copy(out_ref, src_hbm, sem):
        wid = lax.axis_index("s") * _NC + lax.axis_index("c")
        base = wid * rows_per_w
        pltpu.async_copy(
            src_hbm.at[pl.ds(base, rows_per_w)],
            out_ref.at[pl.ds(base, rows_per_w)], sem).wait()
        if tail:
            @pl.when(wid == 0)
            def _():
                pltpu.async_copy(
                    src_hbm.at[pl.ds(rows_per_w * _NW, tail)],
                    out_ref.at[pl.ds(rows_per_w * _NW, tail)], sem).wait()

    return copy


def _make_scatter(d, k):
    @functools.partial(
        pl.kernel,
        mesh=_mesh(),
        compiler_params=_SC_PARAMS,
        scratch_types=[
            pltpu.VMEM((k, _CHUNK), jnp.int32),
            pltpu.VMEM((k, _CHUNK), jnp.int32),
            pltpu.VMEM((_CHUNK, d), jnp.float32),
            pltpu.SemaphoreType.DMA,
            pltpu.SemaphoreType.DMA,
        ],
    )
    def scatter(out_ref, dst_hbm, src_hbm, new_hbm, didx, sidx, rows, gsem, ssem):
        wid = lax.axis_index("s") * _NC + lax.axis_index("c")
        pltpu.sync_copy(dst_hbm.at[wid], didx)
        pltpu.sync_copy(src_hbm.at[wid], sidx)
        for j in range(k):
            pltpu.async_copy(new_hbm.at[sidx.at[j]], rows, gsem).wait()
            pltpu.async_copy(rows, out_ref.at[didx.at[j]], ssem).wait()

    return scatter


def kernel(mat_orig, indices, mat_new):
    m, d = mat_orig.shape
    b = indices.shape[0]
    k = b // (_NW * _CHUNK)

    idx = indices.astype(jnp.int32)
    src = _canonicalize_duplicates(idx)
    dst3 = idx.reshape(_NW, k, _CHUNK)
    src3 = src.reshape(_NW, k, _CHUNK)

    ref = jax.new_ref(lax.empty((m, d), jnp.float32))
    _make_copy(m, d)(ref, mat_orig)
    _make_scatter(d, k)(ref, dst3, src3, mat_new)
    return jax.freeze(ref)


# speedup vs baseline: 11.0652x; 5.8704x over previous
import functools

import jax
import jax.numpy as jnp
from jax import lax
from jax.experimental import pallas as pl
from jax.experimental.pallas import tpu as pltpu
from jax.experimental.pallas import tpu_sc as plsc

_NC = 2
_NS = 16
_NW = _NC * _NS
_CHUNK = 128

_SC_PARAMS = pltpu.CompilerParams(
    use_tc_tiling_on_sc=False, needs_layout_passes=False)


def _canonicalize_duplicates(idx):
    b = idx.shape[0]
    order = jnp.argsort(idx, stable=True)
    si = idx[order]
    pos = jnp.arange(b, dtype=jnp.int32)
    is_end = jnp.concatenate([si[1:] != si[:-1], jnp.ones((1,), jnp.bool_)])
    winner_sorted = lax.associative_scan(
        jnp.minimum, jnp.where(is_end, pos, b), reverse=True)
    winner_orig = order[winner_sorted].astype(jnp.int32)
    src = jnp.zeros((b,), jnp.int32).at[order].set(winner_orig)
    return order, si, src[order]


def _mesh():
    return plsc.VectorSubcoreMesh(
        core_axis_name="c", subcore_axis_name="s",
        num_cores=_NC, num_subcores=_NS)


def _make_fused(m, d, b, rows_per_w, copy_chunk, cap_chunks):
    n_copy = rows_per_w // copy_chunk
    tail = m - rows_per_w * _NW

    @functools.partial(
        pl.kernel,
        mesh=_mesh(),
        compiler_params=_SC_PARAMS,
        out_type=jax.ShapeDtypeStruct((m, d), jnp.float32),
        scratch_types=[
            pltpu.VMEM((copy_chunk, d), jnp.float32),
            pltpu.VMEM((copy_chunk, d), jnp.float32),
            pltpu.VMEM((_CHUNK,), jnp.int32),
            pltpu.VMEM((_CHUNK,), jnp.int32),
            pltpu.VMEM((_CHUNK, d), jnp.float32),
            pltpu.VMEM((_NW,), jnp.int32),
            pltpu.SemaphoreType.DMA,
            pltpu.SemaphoreType.DMA,
            pltpu.SemaphoreType.DMA,
            pltpu.SemaphoreType.DMA,
            pltpu.SemaphoreType.DMA,
            pltpu.SemaphoreType.DMA,
        ],
    )
    def fused(orig_hbm, dst_hbm, srow_hbm, new_hbm, ncnk_hbm, out_ref,
              buf0, buf1, didx, sidx, rows, cnt,
              rs0, rs1, ws0, ws1, gsem, ssem):
        wid = lax.axis_index("s") * _NC + lax.axis_index("c")
        base = wid * rows_per_w
        bufs = (buf0, buf1)
        rsems = (rs0, rs1)
        wsems = (ws0, ws1)

        def rd(c):
            return pltpu.make_async_copy(
                orig_hbm.at[pl.ds(base + c * copy_chunk, copy_chunk)],
                bufs[c % 2], rsems[c % 2])

        def wr(c):
            return pltpu.make_async_copy(
                bufs[c % 2],
                out_ref.at[pl.ds(base + c * copy_chunk, copy_chunk)],
                wsems[c % 2])

        rd(0).start()
        for c in range(n_copy):
            if c + 1 < n_copy:
                if c >= 1:
                    wr(c - 1).wait()
                rd(c + 1).start()
            rd(c).wait()
            wr(c).start()
        if n_copy >= 2:
            wr(n_copy - 2).wait()
        wr(n_copy - 1).wait()

        if tail:
            @pl.when(wid == _NW - 1)
            def _():
                t = pltpu.make_async_copy(
                    orig_hbm.at[pl.ds(rows_per_w * _NW, tail)],
                    bufs[0].at[pl.ds(0, tail)], rsems[0])
                t.start()
                t.wait()
                t2 = pltpu.make_async_copy(
                    bufs[0].at[pl.ds(0, tail)],
                    out_ref.at[pl.ds(rows_per_w * _NW, tail)], wsems[0])
                t2.start()
                t2.wait()

        pltpu.sync_copy(ncnk_hbm, cnt)
        lane = wid % 16
        half = wid // 16
        c_lo = cnt[pl.ds(0, 16)]
        c_hi = cnt[pl.ds(16, 16)]
        sel = jnp.where(
            jnp.full((16,), half == 0, dtype=jnp.bool_), c_lo, c_hi)
        lanes = lax.iota(jnp.int32, 16)
        my_n = jnp.sum(jnp.where(lanes == lane, sel, 0))

        @pl.loop(0, my_n)
        def _(c):
            pltpu.sync_copy(dst_hbm.at[wid].at[c], didx)
            pltpu.sync_copy(srow_hbm.at[wid].at[c], sidx)
            pltpu.async_copy(new_hbm.at[sidx], rows, gsem).wait()
            pltpu.async_copy(rows, out_ref.at[didx], ssem).wait()

    return fused


def kernel(mat_orig, indices, mat_new):
    m, d = mat_orig.shape
    b = indices.shape[0]
    rows_per_w = (m // _NW) // 8 * 8
    copy_chunk = 868
    assert rows_per_w % copy_chunk == 0
    cap_chunks = b // _CHUNK

    idx = indices.astype(jnp.int32)
    order, si, ssrc = _canonicalize_duplicates(idx)

    owner = jnp.minimum(si // rows_per_w, _NW - 1).astype(jnp.int32)
    bounds = (jnp.arange(_NW, dtype=jnp.int32) * rows_per_w)
    pstart = jnp.searchsorted(si, bounds, side="left").astype(jnp.int32)
    pend = jnp.concatenate([pstart[1:], jnp.full((1,), b, jnp.int32)])
    counts = pend - pstart
    ncnk = -(-counts // _CHUNK)
    first = jnp.minimum(pstart, b - 1)
    fill_dst = si[first]
    fill_src = ssrc[first]

    cap = cap_chunks * _CHUNK
    init_dst = jnp.repeat(fill_dst, cap).reshape(_NW * cap)
    init_src = jnp.repeat(fill_src, cap).reshape(_NW * cap)
    slot = owner * cap + (jnp.arange(b, dtype=jnp.int32) - pstart[owner])
    pdst = init_dst.at[slot].set(si).reshape(_NW, cap_chunks, _CHUNK)
    psrc = init_src.at[slot].set(ssrc).reshape(_NW, cap_chunks, _CHUNK)

    fused = _make_fused(m, d, b, rows_per_w, copy_chunk, cap_chunks)
    return fused(mat_orig, pdst, psrc, mat_new, ncnk)


# speedup vs baseline: 11.5206x; 1.0412x over previous
import functools

import jax
import jax.numpy as jnp
from jax import lax
from jax.experimental import pallas as pl
from jax.experimental.pallas import tpu as pltpu
from jax.experimental.pallas import tpu_sc as plsc

_NC = 2
_NS = 16
_NW = _NC * _NS
_CHUNK = 128

_SC_PARAMS = pltpu.CompilerParams(
    use_tc_tiling_on_sc=False, needs_layout_passes=False)


def _mesh():
    return plsc.VectorSubcoreMesh(
        core_axis_name="c", subcore_axis_name="s",
        num_cores=_NC, num_subcores=_NS)


def _make_fused(m, d, b, rows_per_w, copy_chunk):
    n_copy = rows_per_w // copy_chunk
    tail = m - rows_per_w * _NW
    cap_chunks = b // _CHUNK

    @functools.partial(
        pl.kernel,
        mesh=_mesh(),
        compiler_params=_SC_PARAMS,
        out_type=jax.ShapeDtypeStruct((m, d), jnp.float32),
        scratch_types=[
            pltpu.VMEM((copy_chunk, d), jnp.float32),
            pltpu.VMEM((copy_chunk, d), jnp.float32),
            pltpu.VMEM((_CHUNK,), jnp.int32),
            pltpu.VMEM((_CHUNK,), jnp.int32),
            pltpu.VMEM((_CHUNK, d), jnp.float32),
            pltpu.VMEM((16,), jnp.int32),
            pltpu.SemaphoreType.DMA,
            pltpu.SemaphoreType.DMA,
            pltpu.SemaphoreType.DMA,
            pltpu.SemaphoreType.DMA,
            pltpu.SemaphoreType.DMA,
            pltpu.SemaphoreType.DMA,
        ],
    )
    def fused(orig_hbm, dst_hbm, srow_hbm, new_hbm, ncnk_hbm, out_ref,
              buf0, buf1, didx, sidx, rows, cnt,
              rs0, rs1, ws0, ws1, gsem, ssem):
        core = lax.axis_index("c")
        sub = lax.axis_index("s")
        wid = core * _NS + sub
        base = wid * rows_per_w
        bufs = (buf0, buf1)
        rsems = (rs0, rs1)
        wsems = (ws0, ws1)

        def rd(c):
            return pltpu.make_async_copy(
                orig_hbm.at[pl.ds(base + c * copy_chunk, copy_chunk)],
                bufs[c % 2], rsems[c % 2])

        def wr(c):
            return pltpu.make_async_copy(
                bufs[c % 2],
                out_ref.at[pl.ds(base + c * copy_chunk, copy_chunk)],
                wsems[c % 2])

        rd(0).start()
        for c in range(n_copy):
            if c + 1 < n_copy:
                if c >= 1:
                    wr(c - 1).wait()
                rd(c + 1).start()
            rd(c).wait()
            wr(c).start()
        if n_copy >= 2:
            wr(n_copy - 2).wait()
        wr(n_copy - 1).wait()

        if tail:
            @pl.when(wid == _NW - 1)
            def _():
                t = pltpu.make_async_copy(
                    orig_hbm.at[pl.ds(rows_per_w * _NW, tail)],
                    bufs[0].at[pl.ds(0, tail)], rsems[0])
                t.start()
                t.wait()
                t2 = pltpu.make_async_copy(
                    bufs[0].at[pl.ds(0, tail)],
                    out_ref.at[pl.ds(rows_per_w * _NW, tail)], wsems[0])
                t2.start()
                t2.wait()

        plsc.subcore_barrier()

        pltpu.sync_copy(ncnk_hbm, cnt)
        lanes = lax.iota(jnp.int32, 16)
        my_n = jnp.sum(jnp.where(lanes == core, cnt[...], 0))

        @pl.loop(sub, my_n, step=_NS)
        def _(j):
            pltpu.sync_copy(dst_hbm.at[core].at[j], didx)
            pltpu.sync_copy(srow_hbm.at[core].at[j], sidx)
            pltpu.async_copy(new_hbm.at[sidx], rows, gsem).wait()
            pltpu.async_copy(rows, out_ref.at[didx], ssem).wait()

    return fused


def kernel(mat_orig, indices, mat_new):
    m, d = mat_orig.shape
    b = indices.shape[0]
    rows_per_w = (m // _NW) // 8 * 8
    copy_chunk = 868
    assert rows_per_w % copy_chunk == 0

    idx = indices.astype(jnp.int32)
    pos = jnp.arange(b, dtype=jnp.int32)

    wpos = jnp.full((m,), -1, jnp.int32).at[idx].max(pos)
    src = wpos[idx]

    half_rows = _NS * rows_per_w
    g = (idx >= half_rows).astype(jnp.int32)
    r1 = jnp.cumsum(g)
    pig = jnp.where(g == 0, (pos + 1) - r1, r1) - 1
    slot = g * b + pig
    n1 = r1[-1]
    counts = jnp.stack([b - n1, n1])

    first = jnp.stack([jnp.argmax(g == 0), jnp.argmax(g == 1)]).astype(jnp.int32)
    fill_dst = idx[first]
    fill_src = src[first]
    pdst = jnp.repeat(fill_dst, b).at[slot].set(idx).reshape(_NC, b // _CHUNK, _CHUNK)
    psrc = jnp.repeat(fill_src, b).at[slot].set(src).reshape(_NC, b // _CHUNK, _CHUNK)

    ncnk = jnp.zeros((16,), jnp.int32).at[jnp.arange(_NC)].set(
        -(-counts // _CHUNK))

    fused = _make_fused(m, d, b, rows_per_w, copy_chunk)
    return fused(mat_orig, pdst, psrc, mat_new, ncnk)


# speedup vs baseline: 11.9686x; 1.0389x over previous
import functools

import jax
import jax.numpy as jnp
from jax import lax
from jax.experimental import pallas as pl
from jax.experimental.pallas import tpu as pltpu
from jax.experimental.pallas import tpu_sc as plsc

_NC = 2
_NS = 16
_NW = _NC * _NS
_CHUNK = 128
_SENT = -1

_SC_PARAMS = pltpu.CompilerParams(
    use_tc_tiling_on_sc=False, needs_layout_passes=False)


def _mesh():
    return plsc.VectorSubcoreMesh(
        core_axis_name="c", subcore_axis_name="s",
        num_cores=_NC, num_subcores=_NS)


def _make_fused(m, d, b, rows_per_w, copy_chunk):
    n_copy = rows_per_w // copy_chunk
    tail = m - rows_per_w * _NW
    n_chunks = b // _CHUNK
    half = _NS * rows_per_w

    @functools.partial(
        pl.kernel,
        mesh=_mesh(),
        compiler_params=_SC_PARAMS,
        out_type=jax.ShapeDtypeStruct((m, d), jnp.float32),
        scratch_types=[
            pltpu.VMEM((copy_chunk, d), jnp.float32),
            pltpu.VMEM((copy_chunk, d), jnp.float32),
            pltpu.VMEM((_CHUNK,), jnp.int32),
            pltpu.VMEM((_CHUNK,), jnp.int32),
            pltpu.VMEM((_CHUNK,), jnp.int32),
            pltpu.VMEM((_CHUNK, d), jnp.float32),
            pltpu.SemaphoreType.DMA,
            pltpu.SemaphoreType.DMA,
            pltpu.SemaphoreType.DMA,
            pltpu.SemaphoreType.DMA,
            pltpu.SemaphoreType.DMA,
            pltpu.SemaphoreType.DMA,
        ],
    )
    def fused(orig_hbm, idx_hbm, wpos_hbm, new_hbm, out_ref,
              buf0, buf1, didx, wp, midx, rows,
              rs0, rs1, ws0, ws1, gsem, ssem):
        core = lax.axis_index("c")
        sub = lax.axis_index("s")
        wid = core * _NS + sub
        base = wid * rows_per_w
        bufs = (buf0, buf1)
        rsems = (rs0, rs1)
        wsems = (ws0, ws1)

        def rd(c):
            return pltpu.make_async_copy(
                orig_hbm.at[pl.ds(base + c * copy_chunk, copy_chunk)],
                bufs[c % 2], rsems[c % 2])

        def wr(c):
            return pltpu.make_async_copy(
                bufs[c % 2],
                out_ref.at[pl.ds(base + c * copy_chunk, copy_chunk)],
                wsems[c % 2])

        rd(0).start()
        for c in range(n_copy):
            if c + 1 < n_copy:
                if c >= 1:
                    wr(c - 1).wait()
                rd(c + 1).start()
            rd(c).wait()
            wr(c).start()
        if n_copy >= 2:
            wr(n_copy - 2).wait()
        wr(n_copy - 1).wait()

        if tail:
            @pl.when(wid == _NW - 1)
            def _():
                t = pltpu.make_async_copy(
                    orig_hbm.at[pl.ds(rows_per_w * _NW, tail)],
                    bufs[0].at[pl.ds(0, tail)], rsems[0])
                t.start()
                t.wait()
                t2 = pltpu.make_async_copy(
                    bufs[0].at[pl.ds(0, tail)],
                    out_ref.at[pl.ds(rows_per_w * _NW, tail)], wsems[0])
                t2.start()
                t2.wait()

        plsc.subcore_barrier()

        lo = core * half
        hi = half + core * (m - half)

        for k in range(n_chunks // _NS):
            j = sub + k * _NS
            pltpu.sync_copy(idx_hbm.at[j], didx)
            pltpu.async_copy(wpos_hbm.at[didx], wp, gsem).wait()
            for g in range(_CHUNK // 16):
                dv = didx[pl.ds(g * 16, 16)]
                wv = wp[pl.ds(g * 16, 16)]
                mypos = j * _CHUNK + g * 16 + lax.iota(jnp.int32, 16)
                keep = (wv == mypos) & (dv >= lo) & (dv < hi)
                midx[pl.ds(g * 16, 16)] = jnp.where(keep, dv, _SENT)
            pltpu.sync_copy(new_hbm.at[pl.ds(j * _CHUNK, _CHUNK)], rows)
            pltpu.async_copy(
                rows, out_ref.at[plsc.Indices(midx, ignored_value=_SENT)],
                ssem).wait()

    return fused


def kernel(mat_orig, indices, mat_new):
    m, d = mat_orig.shape
    b = indices.shape[0]
    rows_per_w = (m // _NW) // 8 * 8
    copy_chunk = 868
    assert rows_per_w % copy_chunk == 0

    idx = indices.astype(jnp.int32)
    pos = jnp.arange(b, dtype=jnp.int32)
    wpos = jnp.full((m,), -1, jnp.int32).at[idx].max(pos)
    idx2d = idx.reshape(b // _CHUNK, _CHUNK)

    fused = _make_fused(m, d, b, rows_per_w, copy_chunk)
    return fused(mat_orig, idx2d, wpos, mat_new)
